# Initial kernel scaffold; baseline (speedup 1.0000x reference)
import jax
import jax.numpy as jnp
from jax.experimental import pallas as pl


def kernel(drug_feat, dis_feat, edge_index_r1, edge_index_r2, att, basis, W_ufc, b_ufc):
    raise NotImplementedError("write your pallas kernel here")


# speedup vs baseline: 2.6908x; 2.6908x over previous
import jax
import jax.numpy as jnp
from jax import lax
from jax.experimental import pallas as pl
from jax.experimental.pallas import tpu as pltpu
from jax.experimental.pallas import tpu_sc as plsc

N_NODE = 10000
E = 160000
BASIS_UNITS = 4
MSG_RED = 64
MSG = 192
MSGH = 96
MSG_FULL = 384
OUT_UNITS = 256

NC = 2
NS = 16
NW = NC * NS
EPC = E // NC
EPW = E // NW
CHUNK = 128
NFULL = EPW // CHUNK
TAIL = EPW - NFULL * CHUNK
ROWS_PER_S = N_NODE // NS
GCH = 80
NGCH = N_NODE // GCH

def _sc_mesh():
    return plsc.VectorSubcoreMesh(core_axis_name="c", subcore_axis_name="s",
                                  num_cores=NC, num_subcores=NS)


def _wfull_body(att_ref, basis_ref, out_ref):
    a = att_ref[...]
    b = basis_ref[...]
    for r in range(2):
        acc = a[r, 0] * b[0]
        for k in range(1, BASIS_UNITS):
            acc = acc + a[r, k] * b[k]
        out_ref[r] = acc


def _wfull(att, basis):
    nblk = 10
    blk = N_NODE // nblk
    return pl.pallas_call(
        _wfull_body,
        grid=(nblk,),
        in_specs=[
            pl.BlockSpec((2, BASIS_UNITS), lambda i: (0, 0)),
            pl.BlockSpec((BASIS_UNITS, blk, MSG_RED), lambda i: (0, i, 0)),
        ],
        out_specs=pl.BlockSpec((2, blk, MSG_RED), lambda i: (0, i, 0)),
        out_shape=jax.ShapeDtypeStruct((2, N_NODE, MSG_RED), jnp.float32),
    )(att, basis)


def _pre_body(s1, d1, s2, d2,
              dc0, dc1, dc2, ic0, ic1, ic2,
              w0, w1, ones8, zeros8,
              degd_p, degi_p, g_drug, g_dis,
              dacc, iacc, idx_v, idx_v8, ones_v, gbuf, gidx, sem):
    c = lax.axis_index("c")
    s = lax.axis_index("s")
    base_ws = c * EPC + s * EPW
    row0 = s * ROWS_PER_S

    pltpu.sync_copy(zeros8, dacc.at[pl.ds(row0, ROWS_PER_S)])
    pltpu.sync_copy(zeros8, iacc.at[pl.ds(row0, ROWS_PER_S)])
    pltpu.sync_copy(ones8, ones_v)
    plsc.subcore_barrier()

    for stream, acc in ((s1, dacc), (s2, dacc), (d1, iacc), (d2, iacc)):
        def deg_chunk(j, stream=stream, acc=acc):
            b = pl.multiple_of(base_ws + j * CHUNK, 8)
            pltpu.sync_copy(stream.at[pl.ds(b, CHUNK)], idx_v)
            pltpu.sync_copy(ones_v, acc.at[idx_v], add=True)
        lax.fori_loop(0, NFULL, lambda j, _, f=deg_chunk: (f(j), 0)[1], 0)
        bt = base_ws + NFULL * CHUNK
        pltpu.sync_copy(stream.at[pl.ds(bt, TAIL)], idx_v8)
        pltpu.sync_copy(ones_v.at[pl.ds(0, TAIL)], acc.at[idx_v8], add=True)

    plsc.subcore_barrier()
    pltpu.sync_copy(dacc.at[pl.ds(row0, ROWS_PER_S)], degd_p.at[c, s])
    pltpu.sync_copy(iacc.at[pl.ds(row0, ROWS_PER_S)], degi_p.at[c, s])

    wid = c * NS + s
    wtabs = (w0, w1)
    for side, cols, gout in ((0, (dc0, dc1, dc2), g_drug),
                             (1, (ic0, ic1, ic2), g_dis)):
        for r in range(2):
            for k in range(3):
                for t in range(4):
                    cid = wid + NW * t

                    @pl.when(cid < NGCH)
                    def _(cid=cid, col=cols[k], w=wtabs[r], gout=gout, r=r, k=k):
                        nb = pl.multiple_of(cid * GCH, 8)
                        pltpu.sync_copy(col.at[pl.ds(nb, GCH)], gidx)
                        pltpu.async_copy(w.at[gidx], gbuf, sem).wait()
                        pltpu.sync_copy(gbuf, gout.at[r, k, pl.ds(nb, GCH)])


def _pre(s1, d1, s2, d2, dcols, icols, w0, w1):
    ones8 = jnp.ones((ROWS_PER_S, 8), jnp.float32)[:CHUNK]
    zeros8 = jnp.zeros((ROWS_PER_S, 8), jnp.float32)
    out_type = (
        jax.ShapeDtypeStruct((NC, NS, ROWS_PER_S, 8), jnp.float32),
        jax.ShapeDtypeStruct((NC, NS, ROWS_PER_S, 8), jnp.float32),
        jax.ShapeDtypeStruct((2, 3, N_NODE, MSG_RED), jnp.float32),
        jax.ShapeDtypeStruct((2, 3, N_NODE, MSG_RED), jnp.float32),
    )
    scratch = [
        pltpu.VMEM_SHARED((N_NODE, 8), jnp.float32),
        pltpu.VMEM_SHARED((N_NODE, 8), jnp.float32),
        pltpu.VMEM((CHUNK,), jnp.int32),
        pltpu.VMEM((TAIL,), jnp.int32),
        pltpu.VMEM((CHUNK, 8), jnp.float32),
        pltpu.VMEM((GCH, MSG_RED), jnp.float32),
        pltpu.VMEM((GCH,), jnp.int32),
        pltpu.SemaphoreType.DMA,
    ]
    f = pl.kernel(_pre_body, out_type=out_type, mesh=_sc_mesh(),
                  scratch_types=scratch,
                  compiler_params=pltpu.CompilerParams(use_tc_tiling_on_sc=False))
    return f(s1, d1, s2, d2, dcols[0], dcols[1], dcols[2],
             icols[0], icols[1], icols[2], w0, w1, ones8, zeros8)


def _scale_body(degd_ref, degi_ref, gd_ref, gi_ref,
                f1d0_ref, f1d1_ref, f2d0_ref, f2d1_ref,
                f1i0_ref, f1i1_ref, f2i0_ref, f2i1_ref, cd_ref, ci_ref):
    dd = degd_ref[0, :, 0] + degd_ref[1, :, 0]
    di = degi_ref[0, :, 0] + degi_ref[1, :, 0]
    cd = lax.rsqrt(jnp.maximum(dd, 1.0))[:, None]
    ci = lax.rsqrt(jnp.maximum(di, 1.0))[:, None]
    cd_ref[...] = cd
    ci_ref[...] = ci
    halves = (((f1d0_ref, f1d1_ref), (f1i0_ref, f1i1_ref)),
              ((f2d0_ref, f2d1_ref), (f2i0_ref, f2i1_ref)))
    for r in range(2):
        fd = jnp.concatenate(
            [gd_ref[r, 0], gd_ref[r, 1], gd_ref[r, 2]], axis=1) * cd
        fi = jnp.concatenate(
            [gi_ref[r, 0], gi_ref[r, 1], gi_ref[r, 2]], axis=1) * ci
        halves[r][0][0][...] = fd[:, :MSGH]
        halves[r][0][1][...] = fd[:, MSGH:]
        halves[r][1][0][...] = fi[:, :MSGH]
        halves[r][1][1][...] = fi[:, MSGH:]


def _scale(degd_p, degi_p, g_drug, g_dis):
    nblk = 10
    blk = N_NODE // nblk
    out_type = tuple(
        jax.ShapeDtypeStruct((N_NODE, MSGH), jnp.float32) for _ in range(8)
    ) + (jax.ShapeDtypeStruct((N_NODE, 1), jnp.float32),
         jax.ShapeDtypeStruct((N_NODE, 1), jnp.float32))
    deg_spec = pl.BlockSpec((NC, blk, 8), lambda i: (0, i, 0))
    g_spec = pl.BlockSpec((2, 3, blk, MSG_RED), lambda i: (0, 0, i, 0))
    f_spec = pl.BlockSpec((blk, MSGH), lambda i: (i, 0))
    c_spec = pl.BlockSpec((blk, 1), lambda i: (i, 0))
    return pl.pallas_call(
        _scale_body,
        grid=(nblk,),
        in_specs=[deg_spec, deg_spec, g_spec, g_spec],
        out_specs=[f_spec] * 8 + [c_spec, c_spec],
        out_shape=out_type,
    )(degd_p, degi_p, g_drug, g_dis)


def _conv_body(f1d0, f1d1, f2d0, f2d1, f1i0, f1i1, f2i0, f2i1,
               s1, d1, s2, d2, zeros125,
               rst_p, acc, sidx, didx, sidx8, didx8, msg, msg8, zbuf, sem):
    c = lax.axis_index("c")
    s = lax.axis_index("s")
    base_ws = c * EPC + s * EPW
    row0 = s * ROWS_PER_S

    pltpu.sync_copy(zeros125, zbuf)

    convs = (((f1d0, f1d1), s1, d1), ((f2d0, f2d1), s2, d2),
             ((f1i0, f1i1), d1, s1), ((f2i0, f2i1), d2, s2))
    for ci, (ftabs, src, dst) in enumerate(convs):
        for p in range(2):
            ftab = ftabs[p]
            for j in range(5):
                pltpu.sync_copy(zbuf, acc.at[pl.ds(row0 + j * 125, 125)])
            plsc.subcore_barrier()

            def edge_chunk(j, ftab=ftab, src=src, dst=dst):
                b = pl.multiple_of(base_ws + j * CHUNK, 8)
                pltpu.sync_copy(src.at[pl.ds(b, CHUNK)], sidx)
                pltpu.sync_copy(dst.at[pl.ds(b, CHUNK)], didx)
                pltpu.async_copy(ftab.at[sidx], msg, sem).wait()
                pltpu.sync_copy(msg, acc.at[didx], add=True)
            lax.fori_loop(0, NFULL, lambda j, _, f=edge_chunk: (f(j), 0)[1], 0)
            bt = base_ws + NFULL * CHUNK
            pltpu.sync_copy(src.at[pl.ds(bt, TAIL)], sidx8)
            pltpu.sync_copy(dst.at[pl.ds(bt, TAIL)], didx8)
            pltpu.async_copy(ftab.at[sidx8], msg8, sem).wait()
            pltpu.sync_copy(msg8, acc.at[didx8], add=True)

            plsc.subcore_barrier()
            pltpu.sync_copy(acc.at[pl.ds(row0, ROWS_PER_S)],
                            rst_p.at[ci, p, c, s])
            plsc.subcore_barrier()


def _conv(ftabs, s1, d1, s2, d2):
    zeros125 = jnp.zeros((125, MSGH), jnp.float32)
    out_type = jax.ShapeDtypeStruct((4, 2, NC, NS, ROWS_PER_S, MSGH),
                                    jnp.float32)
    scratch = [
        pltpu.VMEM_SHARED((N_NODE, MSGH), jnp.float32),
        pltpu.VMEM((CHUNK,), jnp.int32),
        pltpu.VMEM((CHUNK,), jnp.int32),
        pltpu.VMEM((TAIL,), jnp.int32),
        pltpu.VMEM((TAIL,), jnp.int32),
        pltpu.VMEM((CHUNK, MSGH), jnp.float32),
        pltpu.VMEM((TAIL, MSGH), jnp.float32),
        pltpu.VMEM((125, MSGH), jnp.float32),
        pltpu.SemaphoreType.DMA,
    ]
    f = pl.kernel(_conv_body, out_type=out_type, mesh=_sc_mesh(),
                  scratch_types=scratch,
                  compiler_params=pltpu.CompilerParams(use_tc_tiling_on_sc=False))
    return f(*ftabs, s1, d1, s2, d2, zeros125)


def _proj_body(pa0_ref, pa1_ref, pb0_ref, pb1_ref, c_ref,
               w1_ref, w2_ref, b_ref, out_ref):
    cc = c_ref[...]
    h1 = jnp.concatenate([pa0_ref[0] + pa0_ref[1],
                          pa1_ref[0] + pa1_ref[1]], axis=1) * cc
    h2 = jnp.concatenate([pb0_ref[0] + pb0_ref[1],
                          pb1_ref[0] + pb1_ref[1]], axis=1) * cc
    out_ref[...] = (
        jnp.dot(h1, w1_ref[...], preferred_element_type=jnp.float32)
        + jnp.dot(h2, w2_ref[...], preferred_element_type=jnp.float32)
        + b_ref[...])


def _proj(pa0, pa1, pb0, pb1, c_side, w1, w2, b2d):
    nblk = 10
    blk = N_NODE // nblk
    p_spec = pl.BlockSpec((NC, blk, MSGH), lambda i: (0, i, 0))
    return pl.pallas_call(
        _proj_body,
        grid=(nblk,),
        in_specs=[
            p_spec, p_spec, p_spec, p_spec,
            pl.BlockSpec((blk, 1), lambda i: (i, 0)),
            pl.BlockSpec((MSG, OUT_UNITS), lambda i: (0, 0)),
            pl.BlockSpec((MSG, OUT_UNITS), lambda i: (0, 0)),
            pl.BlockSpec((1, OUT_UNITS), lambda i: (0, 0)),
        ],
        out_specs=pl.BlockSpec((blk, OUT_UNITS), lambda i: (i, 0)),
        out_shape=jax.ShapeDtypeStruct((N_NODE, OUT_UNITS), jnp.float32),
    )(pa0, pa1, pb0, pb1, c_side, w1, w2, b2d)


def kernel(drug_feat, dis_feat, edge_index_r1, edge_index_r2,
           att, basis, W_ufc, b_ufc):
    s1 = edge_index_r1[0].astype(jnp.int32)
    d1 = edge_index_r1[1].astype(jnp.int32)
    s2 = edge_index_r2[0].astype(jnp.int32)
    d2 = edge_index_r2[1].astype(jnp.int32)
    dcols = [drug_feat[:, k].astype(jnp.int32) for k in range(3)]
    icols = [dis_feat[:, k].astype(jnp.int32) for k in range(3)]

    wf = _wfull(att, basis)
    degd_p, degi_p, g_drug, g_dis = _pre(
        s1, d1, s2, d2, dcols, icols, wf[0], wf[1])
    degd_p = degd_p.reshape(NC, N_NODE, 8)
    degi_p = degi_p.reshape(NC, N_NODE, 8)
    *ftabs, c_drug, c_dis = _scale(degd_p, degi_p, g_drug, g_dis)
    rst_p = _conv(ftabs, s1, d1, s2, d2)
    rst_p = rst_p.reshape(4, 2, NC, N_NODE, MSGH)

    w1 = W_ufc[:MSG]
    w2 = W_ufc[MSG:]
    b2d = b_ufc[None, :]
    out_drug = _proj(rst_p[2, 0], rst_p[2, 1], rst_p[3, 0], rst_p[3, 1],
                     c_drug, w1, w2, b2d)
    out_dis = _proj(rst_p[0, 0], rst_p[0, 1], rst_p[1, 0], rst_p[1, 1],
                    c_dis, w1, w2, b2d)
    return jnp.concatenate([out_drug, out_dis], axis=0)


# speedup vs baseline: 3.7810x; 1.4052x over previous
import jax
import jax.numpy as jnp
from jax import lax
from jax.experimental import pallas as pl
from jax.experimental.pallas import tpu as pltpu
from jax.experimental.pallas import tpu_sc as plsc

N_NODE = 10000
E = 160000
BASIS_UNITS = 4
MSG_RED = 64
MSG = 192
MSGH = 96
MSG_FULL = 384
OUT_UNITS = 256

NC = 2
NS = 16
NW = NC * NS
EPC = E // NC
EPW = E // NW
CHUNK = 128
NFULL = EPW // CHUNK
TAIL = EPW - NFULL * CHUNK
ROWS_PER_S = N_NODE // NS
GCH = 80
NGCH = N_NODE // GCH

def _sc_mesh():
    return plsc.VectorSubcoreMesh(core_axis_name="c", subcore_axis_name="s",
                                  num_cores=NC, num_subcores=NS)


def _wfull_body(att_ref, basis_ref, out_ref):
    a = att_ref[...]
    b = basis_ref[...]
    for r in range(2):
        acc = a[r, 0] * b[0]
        for k in range(1, BASIS_UNITS):
            acc = acc + a[r, k] * b[k]
        out_ref[r] = acc


def _wfull(att, basis):
    nblk = 10
    blk = N_NODE // nblk
    return pl.pallas_call(
        _wfull_body,
        grid=(nblk,),
        in_specs=[
            pl.BlockSpec((2, BASIS_UNITS), lambda i: (0, 0)),
            pl.BlockSpec((BASIS_UNITS, blk, MSG_RED), lambda i: (0, i, 0)),
        ],
        out_specs=pl.BlockSpec((2, blk, MSG_RED), lambda i: (0, i, 0)),
        out_shape=jax.ShapeDtypeStruct((2, N_NODE, MSG_RED), jnp.float32),
    )(att, basis)


def _pre_body(s1, d1, s2, d2,
              dc0, dc1, dc2, ic0, ic1, ic2,
              w0, w1, ones8, zeros8,
              degd_p, degi_p, g_drug, g_dis,
              dacc, iacc, idx_v, idx_v8, ones_v, gbuf, gidx, sem):
    c = lax.axis_index("c")
    s = lax.axis_index("s")
    base_ws = c * EPC + s * EPW
    row0 = s * ROWS_PER_S

    pltpu.sync_copy(zeros8, dacc.at[pl.ds(row0, ROWS_PER_S)])
    pltpu.sync_copy(zeros8, iacc.at[pl.ds(row0, ROWS_PER_S)])
    pltpu.sync_copy(ones8, ones_v)
    plsc.subcore_barrier()

    for stream, acc in ((s1, dacc), (s2, dacc), (d1, iacc), (d2, iacc)):
        def deg_chunk(j, stream=stream, acc=acc):
            b = pl.multiple_of(base_ws + j * CHUNK, 8)
            pltpu.sync_copy(stream.at[pl.ds(b, CHUNK)], idx_v)
            pltpu.sync_copy(ones_v, acc.at[idx_v], add=True)
        lax.fori_loop(0, NFULL, lambda j, _, f=deg_chunk: (f(j), 0)[1], 0)
        bt = base_ws + NFULL * CHUNK
        pltpu.sync_copy(stream.at[pl.ds(bt, TAIL)], idx_v8)
        pltpu.sync_copy(ones_v.at[pl.ds(0, TAIL)], acc.at[idx_v8], add=True)

    plsc.subcore_barrier()
    pltpu.sync_copy(dacc.at[pl.ds(row0, ROWS_PER_S)], degd_p.at[c, s])
    pltpu.sync_copy(iacc.at[pl.ds(row0, ROWS_PER_S)], degi_p.at[c, s])

    wid = c * NS + s
    wtabs = (w0, w1)
    for side, cols, gout in ((0, (dc0, dc1, dc2), g_drug),
                             (1, (ic0, ic1, ic2), g_dis)):
        for r in range(2):
            for k in range(3):
                for t in range(4):
                    cid = wid + NW * t

                    @pl.when(cid < NGCH)
                    def _(cid=cid, col=cols[k], w=wtabs[r], gout=gout, r=r, k=k):
                        nb = pl.multiple_of(cid * GCH, 8)
                        pltpu.sync_copy(col.at[pl.ds(nb, GCH)], gidx)
                        pltpu.async_copy(w.at[gidx], gbuf, sem).wait()
                        pltpu.sync_copy(gbuf, gout.at[r, k, pl.ds(nb, GCH)])


def _pre(s1, d1, s2, d2, dcols, icols, w0, w1):
    ones8 = jnp.ones((ROWS_PER_S, 8), jnp.float32)[:CHUNK]
    zeros8 = jnp.zeros((ROWS_PER_S, 8), jnp.float32)
    out_type = (
        jax.ShapeDtypeStruct((NC, NS, ROWS_PER_S, 8), jnp.float32),
        jax.ShapeDtypeStruct((NC, NS, ROWS_PER_S, 8), jnp.float32),
        jax.ShapeDtypeStruct((2, 3, N_NODE, MSG_RED), jnp.float32),
        jax.ShapeDtypeStruct((2, 3, N_NODE, MSG_RED), jnp.float32),
    )
    scratch = [
        pltpu.VMEM_SHARED((N_NODE, 8), jnp.float32),
        pltpu.VMEM_SHARED((N_NODE, 8), jnp.float32),
        pltpu.VMEM((CHUNK,), jnp.int32),
        pltpu.VMEM((TAIL,), jnp.int32),
        pltpu.VMEM((CHUNK, 8), jnp.float32),
        pltpu.VMEM((GCH, MSG_RED), jnp.float32),
        pltpu.VMEM((GCH,), jnp.int32),
        pltpu.SemaphoreType.DMA,
    ]
    f = pl.kernel(_pre_body, out_type=out_type, mesh=_sc_mesh(),
                  scratch_types=scratch,
                  compiler_params=pltpu.CompilerParams(use_tc_tiling_on_sc=False))
    return f(s1, d1, s2, d2, dcols[0], dcols[1], dcols[2],
             icols[0], icols[1], icols[2], w0, w1, ones8, zeros8)


def _scale_body(degd_ref, degi_ref, gd_ref, gi_ref,
                f1d0_ref, f1d1_ref, f2d0_ref, f2d1_ref,
                f1i0_ref, f1i1_ref, f2i0_ref, f2i1_ref, cd_ref, ci_ref):
    dd = degd_ref[0, :, 0] + degd_ref[1, :, 0]
    di = degi_ref[0, :, 0] + degi_ref[1, :, 0]
    cd = lax.rsqrt(jnp.maximum(dd, 1.0))[:, None]
    ci = lax.rsqrt(jnp.maximum(di, 1.0))[:, None]
    cd_ref[...] = cd
    ci_ref[...] = ci
    halves = (((f1d0_ref, f1d1_ref), (f1i0_ref, f1i1_ref)),
              ((f2d0_ref, f2d1_ref), (f2i0_ref, f2i1_ref)))
    for r in range(2):
        fd = jnp.concatenate(
            [gd_ref[r, 0], gd_ref[r, 1], gd_ref[r, 2]], axis=1) * cd
        fi = jnp.concatenate(
            [gi_ref[r, 0], gi_ref[r, 1], gi_ref[r, 2]], axis=1) * ci
        halves[r][0][0][...] = fd[:, :MSGH]
        halves[r][0][1][...] = fd[:, MSGH:]
        halves[r][1][0][...] = fi[:, :MSGH]
        halves[r][1][1][...] = fi[:, MSGH:]


def _scale(degd_p, degi_p, g_drug, g_dis):
    nblk = 10
    blk = N_NODE // nblk
    out_type = tuple(
        jax.ShapeDtypeStruct((N_NODE, MSGH), jnp.float32) for _ in range(8)
    ) + (jax.ShapeDtypeStruct((N_NODE, 1), jnp.float32),
         jax.ShapeDtypeStruct((N_NODE, 1), jnp.float32))
    deg_spec = pl.BlockSpec((NC, blk, 8), lambda i: (0, i, 0))
    g_spec = pl.BlockSpec((2, 3, blk, MSG_RED), lambda i: (0, 0, i, 0))
    f_spec = pl.BlockSpec((blk, MSGH), lambda i: (i, 0))
    c_spec = pl.BlockSpec((blk, 1), lambda i: (i, 0))
    return pl.pallas_call(
        _scale_body,
        grid=(nblk,),
        in_specs=[deg_spec, deg_spec, g_spec, g_spec],
        out_specs=[f_spec] * 8 + [c_spec, c_spec],
        out_shape=out_type,
    )(degd_p, degi_p, g_drug, g_dis)


def _conv_body(f1d0, f1d1, f2d0, f2d1, f1i0, f1i1, f2i0, f2i1,
               s1m, s1t, d1m, d1t, s2m, s2t, d2m, d2t, zeros125,
               rst_p, acc, sidx_all, didx_all, sidx8, didx8,
               msga, msgb, msg8, zbuf, sema, semb):
    c = lax.axis_index("c")
    s = lax.axis_index("s")
    w = c * NS + s
    row0 = s * ROWS_PER_S

    pltpu.sync_copy(zeros125, zbuf)

    convs = (((f1d0, f1d1), s1m, s1t, d1m, d1t),
             ((f2d0, f2d1), s2m, s2t, d2m, d2t),
             ((f1i0, f1i1), d1m, d1t, s1m, s1t),
             ((f2i0, f2i1), d2m, d2t, s2m, s2t))
    for ci, (ftabs, sm, st, dm, dt) in enumerate(convs):
        pltpu.sync_copy(sm.at[w], sidx_all)
        pltpu.sync_copy(dm.at[w], didx_all)
        pltpu.sync_copy(st.at[w], sidx8)
        pltpu.sync_copy(dt.at[w], didx8)
        for p in range(2):
            ftab = ftabs[p]
            for j in range(5):
                pltpu.sync_copy(zbuf, acc.at[pl.ds(row0 + j * 125, 125)])
            plsc.subcore_barrier()

            def start_g(j, buf, sem, ftab=ftab):
                pltpu.async_copy(ftab.at[sidx_all.at[j]], buf, sem)

            def wait_g(buf, sem, ftab=ftab):
                pltpu.make_async_copy(ftab.at[sidx_all.at[0]], buf, sem).wait()

            def scat(j, buf):
                pltpu.sync_copy(buf, acc.at[didx_all.at[j]], add=True)

            start_g(0, msga, sema)

            def pair(j2, _):
                ja = 2 * j2
                wait_g(msga, sema)
                start_g(ja + 1, msgb, semb)
                scat(ja, msga)
                wait_g(msgb, semb)
                start_g(ja + 2, msga, sema)
                scat(ja + 1, msgb)
                return 0
            lax.fori_loop(0, (NFULL - 1) // 2, pair, 0)
            wait_g(msga, sema)
            scat(NFULL - 1, msga)
            pltpu.async_copy(ftab.at[sidx8], msg8, sema).wait()
            pltpu.sync_copy(msg8, acc.at[didx8], add=True)

            plsc.subcore_barrier()
            pltpu.sync_copy(acc.at[pl.ds(row0, ROWS_PER_S)],
                            rst_p.at[ci, p, c, s])
            plsc.subcore_barrier()


def _conv(ftabs, s1, d1, s2, d2):
    zeros125 = jnp.zeros((125, MSGH), jnp.float32)

    def split(e):
        m = e.reshape(NW, EPW)
        return (m[:, :NFULL * CHUNK].reshape(NW, NFULL, CHUNK),
                m[:, NFULL * CHUNK:])
    s1m, s1t = split(s1)
    d1m, d1t = split(d1)
    s2m, s2t = split(s2)
    d2m, d2t = split(d2)
    out_type = jax.ShapeDtypeStruct((4, 2, NC, NS, ROWS_PER_S, MSGH),
                                    jnp.float32)
    scratch = [
        pltpu.VMEM_SHARED((N_NODE, MSGH), jnp.float32),
        pltpu.VMEM((NFULL, CHUNK), jnp.int32),
        pltpu.VMEM((NFULL, CHUNK), jnp.int32),
        pltpu.VMEM((TAIL,), jnp.int32),
        pltpu.VMEM((TAIL,), jnp.int32),
        pltpu.VMEM((CHUNK, MSGH), jnp.float32),
        pltpu.VMEM((CHUNK, MSGH), jnp.float32),
        pltpu.VMEM((TAIL, MSGH), jnp.float32),
        pltpu.VMEM((125, MSGH), jnp.float32),
        pltpu.SemaphoreType.DMA,
        pltpu.SemaphoreType.DMA,
    ]
    f = pl.kernel(_conv_body, out_type=out_type, mesh=_sc_mesh(),
                  scratch_types=scratch,
                  compiler_params=pltpu.CompilerParams(use_tc_tiling_on_sc=False))
    return f(*ftabs, s1m, s1t, d1m, d1t, s2m, s2t, d2m, d2t, zeros125)


def _proj_body(pa0_ref, pa1_ref, pb0_ref, pb1_ref, c_ref,
               w1_ref, w2_ref, b_ref, out_ref):
    cc = c_ref[...]
    h1 = jnp.concatenate([pa0_ref[0] + pa0_ref[1],
                          pa1_ref[0] + pa1_ref[1]], axis=1) * cc
    h2 = jnp.concatenate([pb0_ref[0] + pb0_ref[1],
                          pb1_ref[0] + pb1_ref[1]], axis=1) * cc
    out_ref[...] = (
        jnp.dot(h1, w1_ref[...], preferred_element_type=jnp.float32)
        + jnp.dot(h2, w2_ref[...], preferred_element_type=jnp.float32)
        + b_ref[...])


def _proj(pa0, pa1, pb0, pb1, c_side, w1, w2, b2d):
    nblk = 10
    blk = N_NODE // nblk
    p_spec = pl.BlockSpec((NC, blk, MSGH), lambda i: (0, i, 0))
    return pl.pallas_call(
        _proj_body,
        grid=(nblk,),
        in_specs=[
            p_spec, p_spec, p_spec, p_spec,
            pl.BlockSpec((blk, 1), lambda i: (i, 0)),
            pl.BlockSpec((MSG, OUT_UNITS), lambda i: (0, 0)),
            pl.BlockSpec((MSG, OUT_UNITS), lambda i: (0, 0)),
            pl.BlockSpec((1, OUT_UNITS), lambda i: (0, 0)),
        ],
        out_specs=pl.BlockSpec((blk, OUT_UNITS), lambda i: (i, 0)),
        out_shape=jax.ShapeDtypeStruct((N_NODE, OUT_UNITS), jnp.float32),
    )(pa0, pa1, pb0, pb1, c_side, w1, w2, b2d)


def kernel(drug_feat, dis_feat, edge_index_r1, edge_index_r2,
           att, basis, W_ufc, b_ufc):
    s1 = edge_index_r1[0].astype(jnp.int32)
    d1 = edge_index_r1[1].astype(jnp.int32)
    s2 = edge_index_r2[0].astype(jnp.int32)
    d2 = edge_index_r2[1].astype(jnp.int32)
    dcols = [drug_feat[:, k].astype(jnp.int32) for k in range(3)]
    icols = [dis_feat[:, k].astype(jnp.int32) for k in range(3)]

    wf = _wfull(att, basis)
    degd_p, degi_p, g_drug, g_dis = _pre(
        s1, d1, s2, d2, dcols, icols, wf[0], wf[1])
    degd_p = degd_p.reshape(NC, N_NODE, 8)
    degi_p = degi_p.reshape(NC, N_NODE, 8)
    *ftabs, c_drug, c_dis = _scale(degd_p, degi_p, g_drug, g_dis)
    rst_p = _conv(ftabs, s1, d1, s2, d2)
    rst_p = rst_p.reshape(4, 2, NC, N_NODE, MSGH)

    w1 = W_ufc[:MSG]
    w2 = W_ufc[MSG:]
    b2d = b_ufc[None, :]
    out_drug = _proj(rst_p[2, 0], rst_p[2, 1], rst_p[3, 0], rst_p[3, 1],
                     c_drug, w1, w2, b2d)
    out_dis = _proj(rst_p[0, 0], rst_p[0, 1], rst_p[1, 0], rst_p[1, 1],
                    c_dis, w1, w2, b2d)
    return jnp.concatenate([out_drug, out_dis], axis=0)


# speedup vs baseline: 4.2014x; 1.1112x over previous
import jax
import jax.numpy as jnp
from jax import lax
from jax.experimental import pallas as pl
from jax.experimental.pallas import tpu as pltpu
from jax.experimental.pallas import tpu_sc as plsc

N_NODE = 10000
E = 160000
BASIS_UNITS = 4
MSG_RED = 64
MSG = 192
MSGH = 96
MSG_FULL = 384
OUT_UNITS = 256

NC = 2
NS = 16
NW = NC * NS
EPC = E // NC
EPW = E // NW
CHUNK = 128
NFULL = EPW // CHUNK
TAIL = EPW - NFULL * CHUNK
ROWS_PER_S = N_NODE // NS
GCH = 80
NGCH = 128
N_PAD = NGCH * GCH

def _sc_mesh():
    return plsc.VectorSubcoreMesh(core_axis_name="c", subcore_axis_name="s",
                                  num_cores=NC, num_subcores=NS)


def _wfull_body(att_ref, basis_ref, out_ref):
    a = att_ref[...]
    b = basis_ref[...]
    for r in range(2):
        acc = a[r, 0] * b[0]
        for k in range(1, BASIS_UNITS):
            acc = acc + a[r, k] * b[k]
        out_ref[r] = acc


def _wfull(att, basis):
    nblk = 10
    blk = N_NODE // nblk
    return pl.pallas_call(
        _wfull_body,
        grid=(nblk,),
        in_specs=[
            pl.BlockSpec((2, BASIS_UNITS), lambda i: (0, 0)),
            pl.BlockSpec((BASIS_UNITS, blk, MSG_RED), lambda i: (0, i, 0)),
        ],
        out_specs=pl.BlockSpec((2, blk, MSG_RED), lambda i: (0, i, 0)),
        out_shape=jax.ShapeDtypeStruct((2, N_NODE, MSG_RED), jnp.float32),
    )(att, basis)


def _pre_body(s1m, s1t, d1m, d1t, s2m, s2t, d2m, d2t,
              dc0, dc1, dc2, ic0, ic1, ic2,
              w0, w1, ones8, zeros8,
              degd_p, degi_p, g_drug, g_dis,
              dacc, iacc, idxbig, idx_v8, ones_v, gbufa, gbufb, idx4,
              sema, semb):
    c = lax.axis_index("c")
    s = lax.axis_index("s")
    w = c * NS + s
    row0 = s * ROWS_PER_S

    pltpu.sync_copy(zeros8, dacc.at[pl.ds(row0, ROWS_PER_S)])
    pltpu.sync_copy(zeros8, iacc.at[pl.ds(row0, ROWS_PER_S)])
    pltpu.sync_copy(ones8, ones_v)
    plsc.subcore_barrier()

    for sm, st, acc in ((s1m, s1t, dacc), (s2m, s2t, dacc),
                        (d1m, d1t, iacc), (d2m, d2t, iacc)):
        pltpu.sync_copy(sm.at[w], idxbig)
        def deg_chunk(j, acc=acc):
            pltpu.sync_copy(ones_v, acc.at[idxbig.at[j]], add=True)
        lax.fori_loop(0, NFULL, lambda j, _, f=deg_chunk: (f(j), 0)[1], 0)
        pltpu.sync_copy(st.at[w], idx_v8)
        pltpu.sync_copy(ones_v.at[pl.ds(0, TAIL)], acc.at[idx_v8], add=True)

    plsc.subcore_barrier()
    pltpu.sync_copy(dacc.at[pl.ds(row0, ROWS_PER_S)], degd_p.at[c, s])
    pltpu.sync_copy(iacc.at[pl.ds(row0, ROWS_PER_S)], degi_p.at[c, s])

    wtabs = (w0, w1)
    bufs, sems = (gbufa, gbufb), (sema, semb)
    pend = []
    q = 0

    def drain():
        buf, sem, gout, r, k, t = pend.pop()
        pltpu.make_async_copy(wtabs[0].at[idx4.at[0]], buf, sem).wait()
        nb = pl.multiple_of((w + NW * t) * GCH, 8)
        pltpu.sync_copy(buf, gout.at[r, k, pl.ds(nb, GCH)])

    for side, cols, gout in ((0, (dc0, dc1, dc2), g_drug),
                             (1, (ic0, ic1, ic2), g_dis)):
        for k in range(3):
            if pend:
                drain()
            pltpu.sync_copy(cols[k].at[:, w], idx4)
            for r in range(2):
                for t in range(4):
                    b = q % 2
                    pltpu.async_copy(wtabs[r].at[idx4.at[t]], bufs[b],
                                     sems[b])
                    if pend:
                        drain()
                    pend.append((bufs[b], sems[b], gout, r, k, t))
                    q += 1
    drain()


def _pre(edges_split, dcols, icols, w0, w1):
    ones8 = jnp.ones((CHUNK, 8), jnp.float32)
    zeros8 = jnp.zeros((ROWS_PER_S, 8), jnp.float32)
    out_type = (
        jax.ShapeDtypeStruct((NC, NS, ROWS_PER_S, 8), jnp.float32),
        jax.ShapeDtypeStruct((NC, NS, ROWS_PER_S, 8), jnp.float32),
        jax.ShapeDtypeStruct((2, 3, N_PAD, MSG_RED), jnp.float32),
        jax.ShapeDtypeStruct((2, 3, N_PAD, MSG_RED), jnp.float32),
    )
    scratch = [
        pltpu.VMEM_SHARED((N_NODE, 8), jnp.float32),
        pltpu.VMEM_SHARED((N_NODE, 8), jnp.float32),
        pltpu.VMEM((NFULL, CHUNK), jnp.int32),
        pltpu.VMEM((TAIL,), jnp.int32),
        pltpu.VMEM((CHUNK, 8), jnp.float32),
        pltpu.VMEM((GCH, MSG_RED), jnp.float32),
        pltpu.VMEM((GCH, MSG_RED), jnp.float32),
        pltpu.VMEM((4, GCH), jnp.int32),
        pltpu.SemaphoreType.DMA,
        pltpu.SemaphoreType.DMA,
    ]
    f = pl.kernel(_pre_body, out_type=out_type, mesh=_sc_mesh(),
                  scratch_types=scratch,
                  compiler_params=pltpu.CompilerParams(use_tc_tiling_on_sc=False))
    return f(*edges_split, dcols[0], dcols[1], dcols[2],
             icols[0], icols[1], icols[2], w0, w1, ones8, zeros8)


def _scale_body(degd_ref, degi_ref, gd_ref, gi_ref,
                f1d0_ref, f1d1_ref, f2d0_ref, f2d1_ref,
                f1i0_ref, f1i1_ref, f2i0_ref, f2i1_ref, cd_ref, ci_ref):
    dd = degd_ref[0, :, 0] + degd_ref[1, :, 0]
    di = degi_ref[0, :, 0] + degi_ref[1, :, 0]
    cd = lax.rsqrt(jnp.maximum(dd, 1.0))[:, None]
    ci = lax.rsqrt(jnp.maximum(di, 1.0))[:, None]
    cd_ref[...] = cd
    ci_ref[...] = ci
    halves = (((f1d0_ref, f1d1_ref), (f1i0_ref, f1i1_ref)),
              ((f2d0_ref, f2d1_ref), (f2i0_ref, f2i1_ref)))
    for r in range(2):
        fd = jnp.concatenate(
            [gd_ref[r, 0], gd_ref[r, 1], gd_ref[r, 2]], axis=1) * cd
        fi = jnp.concatenate(
            [gi_ref[r, 0], gi_ref[r, 1], gi_ref[r, 2]], axis=1) * ci
        halves[r][0][0][...] = fd[:, :MSGH]
        halves[r][0][1][...] = fd[:, MSGH:]
        halves[r][1][0][...] = fi[:, :MSGH]
        halves[r][1][1][...] = fi[:, MSGH:]


def _scale(degd_p, degi_p, g_drug, g_dis):
    nblk = 10
    blk = N_NODE // nblk
    out_type = tuple(
        jax.ShapeDtypeStruct((N_NODE, MSGH), jnp.float32) for _ in range(8)
    ) + (jax.ShapeDtypeStruct((N_NODE, 1), jnp.float32),
         jax.ShapeDtypeStruct((N_NODE, 1), jnp.float32))
    deg_spec = pl.BlockSpec((NC, blk, 8), lambda i: (0, i, 0))
    g_spec = pl.BlockSpec((2, 3, blk, MSG_RED), lambda i: (0, 0, i, 0))
    f_spec = pl.BlockSpec((blk, MSGH), lambda i: (i, 0))
    c_spec = pl.BlockSpec((blk, 1), lambda i: (i, 0))
    return pl.pallas_call(
        _scale_body,
        grid=(nblk,),
        in_specs=[deg_spec, deg_spec, g_spec, g_spec],
        out_specs=[f_spec] * 8 + [c_spec, c_spec],
        out_shape=out_type,
    )(degd_p, degi_p, g_drug, g_dis)


def _conv_body(f1d0, f1d1, f2d0, f2d1, f1i0, f1i1, f2i0, f2i1,
               s1m, s1t, d1m, d1t, s2m, s2t, d2m, d2t, zeros125,
               rst_p, acc, sidx_all, didx_all, sidx8, didx8,
               msga, msgb, msg8, zbuf, sema, semb):
    c = lax.axis_index("c")
    s = lax.axis_index("s")
    w = c * NS + s
    row0 = s * ROWS_PER_S

    pltpu.sync_copy(zeros125, zbuf)

    convs = (((f1d0, f1d1), s1m, s1t, d1m, d1t),
             ((f2d0, f2d1), s2m, s2t, d2m, d2t),
             ((f1i0, f1i1), d1m, d1t, s1m, s1t),
             ((f2i0, f2i1), d2m, d2t, s2m, s2t))
    for ci, (ftabs, sm, st, dm, dt) in enumerate(convs):
        pltpu.sync_copy(sm.at[w], sidx_all)
        pltpu.sync_copy(dm.at[w], didx_all)
        pltpu.sync_copy(st.at[w], sidx8)
        pltpu.sync_copy(dt.at[w], didx8)
        for p in range(2):
            ftab = ftabs[p]
            for j in range(5):
                pltpu.sync_copy(zbuf, acc.at[pl.ds(row0 + j * 125, 125)])
            plsc.subcore_barrier()

            def start_g(j, buf, sem, ftab=ftab):
                pltpu.async_copy(ftab.at[sidx_all.at[j]], buf, sem)

            def wait_g(buf, sem, ftab=ftab):
                pltpu.make_async_copy(ftab.at[sidx_all.at[0]], buf, sem).wait()

            def scat(j, buf):
                pltpu.sync_copy(buf, acc.at[didx_all.at[j]], add=True)

            start_g(0, msga, sema)

            def pair(j2, _):
                ja = 2 * j2
                wait_g(msga, sema)
                start_g(ja + 1, msgb, semb)
                scat(ja, msga)
                wait_g(msgb, semb)
                start_g(ja + 2, msga, sema)
                scat(ja + 1, msgb)
                return 0
            lax.fori_loop(0, (NFULL - 1) // 2, pair, 0)
            wait_g(msga, sema)
            scat(NFULL - 1, msga)
            pltpu.async_copy(ftab.at[sidx8], msg8, sema).wait()
            pltpu.sync_copy(msg8, acc.at[didx8], add=True)

            plsc.subcore_barrier()
            pltpu.sync_copy(acc.at[pl.ds(row0, ROWS_PER_S)],
                            rst_p.at[ci, p, c, s])
            plsc.subcore_barrier()


def _conv(ftabs, edges_split):
    zeros125 = jnp.zeros((125, MSGH), jnp.float32)
    out_type = jax.ShapeDtypeStruct((4, 2, NC, NS, ROWS_PER_S, MSGH),
                                    jnp.float32)
    scratch = [
        pltpu.VMEM_SHARED((N_NODE, MSGH), jnp.float32),
        pltpu.VMEM((NFULL, CHUNK), jnp.int32),
        pltpu.VMEM((NFULL, CHUNK), jnp.int32),
        pltpu.VMEM((TAIL,), jnp.int32),
        pltpu.VMEM((TAIL,), jnp.int32),
        pltpu.VMEM((CHUNK, MSGH), jnp.float32),
        pltpu.VMEM((CHUNK, MSGH), jnp.float32),
        pltpu.VMEM((TAIL, MSGH), jnp.float32),
        pltpu.VMEM((125, MSGH), jnp.float32),
        pltpu.SemaphoreType.DMA,
        pltpu.SemaphoreType.DMA,
    ]
    f = pl.kernel(_conv_body, out_type=out_type, mesh=_sc_mesh(),
                  scratch_types=scratch,
                  compiler_params=pltpu.CompilerParams(use_tc_tiling_on_sc=False))
    return f(*ftabs, *edges_split, zeros125)


def _proj_body(pa0_ref, pa1_ref, pb0_ref, pb1_ref, c_ref,
               w1_ref, w2_ref, b_ref, out_ref):
    cc = c_ref[...]
    h1 = jnp.concatenate([pa0_ref[0, 0, 0] + pa0_ref[0, 0, 1],
                          pa1_ref[0, 0, 0] + pa1_ref[0, 0, 1]], axis=1) * cc
    h2 = jnp.concatenate([pb0_ref[0, 0, 0] + pb0_ref[0, 0, 1],
                          pb1_ref[0, 0, 0] + pb1_ref[0, 0, 1]], axis=1) * cc
    out_ref[...] = (
        jnp.dot(h1, w1_ref[...], preferred_element_type=jnp.float32)
        + jnp.dot(h2, w2_ref[...], preferred_element_type=jnp.float32)
        + b_ref[...])


def _proj(rst_p, ca, cb, c_side, w1, w2, b2d):
    nblk = 10
    blk = N_NODE // nblk

    def p_spec(ci, p):
        return pl.BlockSpec((1, 1, NC, blk, MSGH),
                            lambda i, ci=ci, p=p: (ci, p, 0, i, 0))
    return pl.pallas_call(
        _proj_body,
        grid=(nblk,),
        in_specs=[
            p_spec(ca, 0), p_spec(ca, 1), p_spec(cb, 0), p_spec(cb, 1),
            pl.BlockSpec((blk, 1), lambda i: (i, 0)),
            pl.BlockSpec((MSG, OUT_UNITS), lambda i: (0, 0)),
            pl.BlockSpec((MSG, OUT_UNITS), lambda i: (0, 0)),
            pl.BlockSpec((1, OUT_UNITS), lambda i: (0, 0)),
        ],
        out_specs=pl.BlockSpec((blk, OUT_UNITS), lambda i: (i, 0)),
        out_shape=jax.ShapeDtypeStruct((N_NODE, OUT_UNITS), jnp.float32),
    )(rst_p, rst_p, rst_p, rst_p, c_side, w1, w2, b2d)


def kernel(drug_feat, dis_feat, edge_index_r1, edge_index_r2,
           att, basis, W_ufc, b_ufc):
    def split(e):
        m = e.astype(jnp.int32).reshape(NW, EPW)
        return (m[:, :NFULL * CHUNK].reshape(NW, NFULL, CHUNK),
                m[:, NFULL * CHUNK:])

    edges_split = []
    for arr in (edge_index_r1, edge_index_r2):
        for row in (0, 1):
            edges_split.extend(split(arr[row]))

    def colpack(feat, k):
        col = jnp.pad(feat[:, k].astype(jnp.int32), (0, N_PAD - N_NODE))
        return col.reshape(4, NW, GCH)
    dcols = [colpack(drug_feat, k) for k in range(3)]
    icols = [colpack(dis_feat, k) for k in range(3)]

    wf = _wfull(att, basis)
    degd_p, degi_p, g_drug, g_dis = _pre(
        edges_split, dcols, icols, wf[0], wf[1])
    degd_p = degd_p.reshape(NC, N_NODE, 8)
    degi_p = degi_p.reshape(NC, N_NODE, 8)
    *ftabs, c_drug, c_dis = _scale(degd_p, degi_p, g_drug, g_dis)
    rst_p = _conv(ftabs, edges_split)
    rst_p = rst_p.reshape(4, 2, NC, N_NODE, MSGH)

    w1 = W_ufc[:MSG]
    w2 = W_ufc[MSG:]
    b2d = b_ufc[None, :]
    out_drug = _proj(rst_p, 2, 3, c_drug, w1, w2, b2d)
    out_dis = _proj(rst_p, 0, 1, c_dis, w1, w2, b2d)
    return jnp.concatenate([out_drug, out_dis], axis=0)


# speedup vs baseline: 4.2480x; 1.0111x over previous
import jax
import jax.numpy as jnp
from jax import lax
from jax.experimental import pallas as pl
from jax.experimental.pallas import tpu as pltpu
from jax.experimental.pallas import tpu_sc as plsc

N_NODE = 10000
E = 160000
BASIS_UNITS = 4
MSG_RED = 64
MSG = 192
MSGH = 96
MSG_FULL = 384
OUT_UNITS = 256

NC = 2
NS = 16
NW = NC * NS
EPC = E // NC
EPW = E // NW
CHUNK = 128
NFULL = EPW // CHUNK
TAIL = EPW - NFULL * CHUNK
ROWS_PER_S = N_NODE // NS
GCH = 80
NGCH = 128
N_PAD = NGCH * GCH

def _sc_mesh():
    return plsc.VectorSubcoreMesh(core_axis_name="c", subcore_axis_name="s",
                                  num_cores=NC, num_subcores=NS)


def _wfull_body(att_ref, basis_ref, out_ref):
    a = att_ref[...]
    b = basis_ref[...]
    for r in range(2):
        acc = a[r, 0] * b[0]
        for k in range(1, BASIS_UNITS):
            acc = acc + a[r, k] * b[k]
        out_ref[r] = acc


def _wfull(att, basis):
    nblk = 10
    blk = N_NODE // nblk
    return pl.pallas_call(
        _wfull_body,
        grid=(nblk,),
        in_specs=[
            pl.BlockSpec((2, BASIS_UNITS), lambda i: (0, 0)),
            pl.BlockSpec((BASIS_UNITS, blk, MSG_RED), lambda i: (0, i, 0)),
        ],
        out_specs=pl.BlockSpec((2, blk, MSG_RED), lambda i: (0, i, 0)),
        out_shape=jax.ShapeDtypeStruct((2, N_NODE, MSG_RED), jnp.float32),
    )(att, basis)


def _pre_body(s1m, s1t, d1m, d1t, s2m, s2t, d2m, d2t,
              dc0, dc1, dc2, ic0, ic1, ic2,
              w0, w1, ones8, zeros8,
              degd_p, degi_p, g_drug, g_dis,
              dacc, iacc, idxbig, idx_v8, ones_v, gbufa, gbufb, idx4,
              sema, semb):
    c = lax.axis_index("c")
    s = lax.axis_index("s")
    w = c * NS + s
    row0 = s * ROWS_PER_S

    pltpu.sync_copy(zeros8, dacc.at[pl.ds(row0, ROWS_PER_S)])
    pltpu.sync_copy(zeros8, iacc.at[pl.ds(row0, ROWS_PER_S)])
    pltpu.sync_copy(ones8, ones_v)
    plsc.subcore_barrier()

    for sm, st, acc in ((s1m, s1t, dacc), (s2m, s2t, dacc),
                        (d1m, d1t, iacc), (d2m, d2t, iacc)):
        pltpu.sync_copy(sm.at[w], idxbig)
        def deg_chunk(j, acc=acc):
            pltpu.sync_copy(ones_v, acc.at[idxbig.at[j]], add=True)
        lax.fori_loop(0, NFULL, lambda j, _, f=deg_chunk: (f(j), 0)[1], 0)
        pltpu.sync_copy(st.at[w], idx_v8)
        pltpu.sync_copy(ones_v.at[pl.ds(0, TAIL)], acc.at[idx_v8], add=True)

    plsc.subcore_barrier()
    pltpu.sync_copy(dacc.at[pl.ds(row0, ROWS_PER_S)], degd_p.at[c, s])
    pltpu.sync_copy(iacc.at[pl.ds(row0, ROWS_PER_S)], degi_p.at[c, s])

    wtabs = (w0, w1)
    bufs, sems = (gbufa, gbufb), (sema, semb)
    pend = []
    q = 0

    def drain():
        buf, sem, gout, r, k, t = pend.pop()
        pltpu.make_async_copy(wtabs[0].at[idx4.at[0]], buf, sem).wait()
        nb = pl.multiple_of((w + NW * t) * GCH, 8)
        pltpu.sync_copy(buf, gout.at[r, k, pl.ds(nb, GCH)])

    for side, cols, gout in ((0, (dc0, dc1, dc2), g_drug),
                             (1, (ic0, ic1, ic2), g_dis)):
        for k in range(3):
            if pend:
                drain()
            pltpu.sync_copy(cols[k].at[:, w], idx4)
            for r in range(2):
                for t in range(4):
                    b = q % 2
                    pltpu.async_copy(wtabs[r].at[idx4.at[t]], bufs[b],
                                     sems[b])
                    if pend:
                        drain()
                    pend.append((bufs[b], sems[b], gout, r, k, t))
                    q += 1
    drain()


def _pre(edges_split, dcols, icols, w0, w1):
    ones8 = jnp.ones((CHUNK, 8), jnp.float32)
    zeros8 = jnp.zeros((ROWS_PER_S, 8), jnp.float32)
    out_type = (
        jax.ShapeDtypeStruct((NC, NS, ROWS_PER_S, 8), jnp.float32),
        jax.ShapeDtypeStruct((NC, NS, ROWS_PER_S, 8), jnp.float32),
        jax.ShapeDtypeStruct((2, 3, N_PAD, MSG_RED), jnp.float32),
        jax.ShapeDtypeStruct((2, 3, N_PAD, MSG_RED), jnp.float32),
    )
    scratch = [
        pltpu.VMEM_SHARED((N_NODE, 8), jnp.float32),
        pltpu.VMEM_SHARED((N_NODE, 8), jnp.float32),
        pltpu.VMEM((NFULL, CHUNK), jnp.int32),
        pltpu.VMEM((TAIL,), jnp.int32),
        pltpu.VMEM((CHUNK, 8), jnp.float32),
        pltpu.VMEM((GCH, MSG_RED), jnp.float32),
        pltpu.VMEM((GCH, MSG_RED), jnp.float32),
        pltpu.VMEM((4, GCH), jnp.int32),
        pltpu.SemaphoreType.DMA,
        pltpu.SemaphoreType.DMA,
    ]
    f = pl.kernel(_pre_body, out_type=out_type, mesh=_sc_mesh(),
                  scratch_types=scratch,
                  compiler_params=pltpu.CompilerParams(use_tc_tiling_on_sc=False))
    return f(*edges_split, dcols[0], dcols[1], dcols[2],
             icols[0], icols[1], icols[2], w0, w1, ones8, zeros8)


def _scale_body(degd_ref, degi_ref, gd_ref, gi_ref,
                f1d0_ref, f1d1_ref, f2d0_ref, f2d1_ref,
                f1i0_ref, f1i1_ref, f2i0_ref, f2i1_ref, cd_ref, ci_ref):
    dd = degd_ref[0, :, 0] + degd_ref[1, :, 0]
    di = degi_ref[0, :, 0] + degi_ref[1, :, 0]
    cd = lax.rsqrt(jnp.maximum(dd, 1.0))[:, None]
    ci = lax.rsqrt(jnp.maximum(di, 1.0))[:, None]
    cd_ref[...] = cd
    ci_ref[...] = ci
    halves = (((f1d0_ref, f1d1_ref), (f1i0_ref, f1i1_ref)),
              ((f2d0_ref, f2d1_ref), (f2i0_ref, f2i1_ref)))
    for r in range(2):
        fd = jnp.concatenate(
            [gd_ref[r, 0], gd_ref[r, 1], gd_ref[r, 2]], axis=1) * cd
        fi = jnp.concatenate(
            [gi_ref[r, 0], gi_ref[r, 1], gi_ref[r, 2]], axis=1) * ci
        halves[r][0][0][...] = fd[:, :MSGH].astype(jnp.bfloat16)
        halves[r][0][1][...] = fd[:, MSGH:].astype(jnp.bfloat16)
        halves[r][1][0][...] = fi[:, :MSGH].astype(jnp.bfloat16)
        halves[r][1][1][...] = fi[:, MSGH:].astype(jnp.bfloat16)


def _scale(degd_p, degi_p, g_drug, g_dis):
    nblk = 10
    blk = N_NODE // nblk
    out_type = tuple(
        jax.ShapeDtypeStruct((N_NODE, MSGH), jnp.bfloat16) for _ in range(8)
    ) + (jax.ShapeDtypeStruct((N_NODE, 1), jnp.float32),
         jax.ShapeDtypeStruct((N_NODE, 1), jnp.float32))
    deg_spec = pl.BlockSpec((NC, blk, 8), lambda i: (0, i, 0))
    g_spec = pl.BlockSpec((2, 3, blk, MSG_RED), lambda i: (0, 0, i, 0))
    f_spec = pl.BlockSpec((blk, MSGH), lambda i: (i, 0))
    c_spec = pl.BlockSpec((blk, 1), lambda i: (i, 0))
    return pl.pallas_call(
        _scale_body,
        grid=(nblk,),
        in_specs=[deg_spec, deg_spec, g_spec, g_spec],
        out_specs=[f_spec] * 8 + [c_spec, c_spec],
        out_shape=out_type,
    )(degd_p, degi_p, g_drug, g_dis)


def _conv_body(f1d0, f1d1, f2d0, f2d1, f1i0, f1i1, f2i0, f2i1,
               s1m, s1t, d1m, d1t, s2m, s2t, d2m, d2t, zeros125,
               rst_p, acc, sidx_all, didx_all, sidx8, didx8,
               msga, msgb, msg8, zbuf, sema, semb):
    c = lax.axis_index("c")
    s = lax.axis_index("s")
    w = c * NS + s
    row0 = s * ROWS_PER_S

    pltpu.sync_copy(zeros125, zbuf)

    convs = (((f1d0, f1d1), s1m, s1t, d1m, d1t),
             ((f2d0, f2d1), s2m, s2t, d2m, d2t),
             ((f1i0, f1i1), d1m, d1t, s1m, s1t),
             ((f2i0, f2i1), d2m, d2t, s2m, s2t))
    for ci, (ftabs, sm, st, dm, dt) in enumerate(convs):
        pltpu.sync_copy(sm.at[w], sidx_all)
        pltpu.sync_copy(dm.at[w], didx_all)
        pltpu.sync_copy(st.at[w], sidx8)
        pltpu.sync_copy(dt.at[w], didx8)
        for p in range(2):
            ftab = ftabs[p]
            for j in range(5):
                pltpu.sync_copy(zbuf, acc.at[pl.ds(row0 + j * 125, 125)])
            plsc.subcore_barrier()

            def start_g(j, buf, sem, ftab=ftab):
                pltpu.async_copy(ftab.at[sidx_all.at[j]], buf, sem)

            def wait_g(buf, sem, ftab=ftab):
                pltpu.make_async_copy(ftab.at[sidx_all.at[0]], buf, sem).wait()

            def scat(j, buf):
                pltpu.sync_copy(buf, acc.at[didx_all.at[j]], add=True)

            start_g(0, msga, sema)

            def pair(j2, _):
                ja = 2 * j2
                wait_g(msga, sema)
                start_g(ja + 1, msgb, semb)
                scat(ja, msga)
                wait_g(msgb, semb)
                start_g(ja + 2, msga, sema)
                scat(ja + 1, msgb)
                return 0
            lax.fori_loop(0, (NFULL - 1) // 2, pair, 0)
            wait_g(msga, sema)
            scat(NFULL - 1, msga)
            pltpu.async_copy(ftab.at[sidx8], msg8, sema).wait()
            pltpu.sync_copy(msg8, acc.at[didx8], add=True)

            plsc.subcore_barrier()
            pltpu.sync_copy(acc.at[pl.ds(row0, ROWS_PER_S)],
                            rst_p.at[ci, p, c, s])
            plsc.subcore_barrier()


def _conv(ftabs, edges_split):
    zeros125 = jnp.zeros((125, MSGH), jnp.bfloat16)
    out_type = jax.ShapeDtypeStruct((4, 2, NC, NS, ROWS_PER_S, MSGH),
                                    jnp.bfloat16)
    scratch = [
        pltpu.VMEM_SHARED((N_NODE, MSGH), jnp.bfloat16),
        pltpu.VMEM((NFULL, CHUNK), jnp.int32),
        pltpu.VMEM((NFULL, CHUNK), jnp.int32),
        pltpu.VMEM((TAIL,), jnp.int32),
        pltpu.VMEM((TAIL,), jnp.int32),
        pltpu.VMEM((CHUNK, MSGH), jnp.bfloat16),
        pltpu.VMEM((CHUNK, MSGH), jnp.bfloat16),
        pltpu.VMEM((TAIL, MSGH), jnp.bfloat16),
        pltpu.VMEM((125, MSGH), jnp.bfloat16),
        pltpu.SemaphoreType.DMA,
        pltpu.SemaphoreType.DMA,
    ]
    f = pl.kernel(_conv_body, out_type=out_type, mesh=_sc_mesh(),
                  scratch_types=scratch,
                  compiler_params=pltpu.CompilerParams(use_tc_tiling_on_sc=False))
    return f(*ftabs, *edges_split, zeros125)


def _proj_body(pa0_ref, pa1_ref, pb0_ref, pb1_ref, c_ref,
               w1_ref, w2_ref, b_ref, out_ref):
    cc = c_ref[...]

    def m(ref):
        return (ref[0, 0, 0].astype(jnp.float32)
                + ref[0, 0, 1].astype(jnp.float32))
    h1 = jnp.concatenate([m(pa0_ref), m(pa1_ref)], axis=1) * cc
    h2 = jnp.concatenate([m(pb0_ref), m(pb1_ref)], axis=1) * cc
    out_ref[...] = (
        jnp.dot(h1, w1_ref[...], preferred_element_type=jnp.float32)
        + jnp.dot(h2, w2_ref[...], preferred_element_type=jnp.float32)
        + b_ref[...])


def _proj(rst_p, ca, cb, c_side, w1, w2, b2d):
    nblk = 10
    blk = N_NODE // nblk

    def p_spec(ci, p):
        return pl.BlockSpec((1, 1, NC, blk, MSGH),
                            lambda i, ci=ci, p=p: (ci, p, 0, i, 0))
    return pl.pallas_call(
        _proj_body,
        grid=(nblk,),
        in_specs=[
            p_spec(ca, 0), p_spec(ca, 1), p_spec(cb, 0), p_spec(cb, 1),
            pl.BlockSpec((blk, 1), lambda i: (i, 0)),
            pl.BlockSpec((MSG, OUT_UNITS), lambda i: (0, 0)),
            pl.BlockSpec((MSG, OUT_UNITS), lambda i: (0, 0)),
            pl.BlockSpec((1, OUT_UNITS), lambda i: (0, 0)),
        ],
        out_specs=pl.BlockSpec((blk, OUT_UNITS), lambda i: (i, 0)),
        out_shape=jax.ShapeDtypeStruct((N_NODE, OUT_UNITS), jnp.float32),
    )(rst_p, rst_p, rst_p, rst_p, c_side, w1, w2, b2d)


def kernel(drug_feat, dis_feat, edge_index_r1, edge_index_r2,
           att, basis, W_ufc, b_ufc):
    def split(e):
        m = e.astype(jnp.int32).reshape(NW, EPW)
        return (m[:, :NFULL * CHUNK].reshape(NW, NFULL, CHUNK),
                m[:, NFULL * CHUNK:])

    edges_split = []
    for arr in (edge_index_r1, edge_index_r2):
        for row in (0, 1):
            edges_split.extend(split(arr[row]))

    def colpack(feat, k):
        col = jnp.pad(feat[:, k].astype(jnp.int32), (0, N_PAD - N_NODE))
        return col.reshape(4, NW, GCH)
    dcols = [colpack(drug_feat, k) for k in range(3)]
    icols = [colpack(dis_feat, k) for k in range(3)]

    wf = _wfull(att, basis)
    degd_p, degi_p, g_drug, g_dis = _pre(
        edges_split, dcols, icols, wf[0], wf[1])
    degd_p = degd_p.reshape(NC, N_NODE, 8)
    degi_p = degi_p.reshape(NC, N_NODE, 8)
    *ftabs, c_drug, c_dis = _scale(degd_p, degi_p, g_drug, g_dis)
    rst_p = _conv(ftabs, edges_split)
    rst_p = rst_p.reshape(4, 2, NC, N_NODE, MSGH)

    w1 = W_ufc[:MSG]
    w2 = W_ufc[MSG:]
    b2d = b_ufc[None, :]
    out_drug = _proj(rst_p, 2, 3, c_drug, w1, w2, b2d)
    out_dis = _proj(rst_p, 0, 1, c_dis, w1, w2, b2d)
    return jnp.concatenate([out_drug, out_dis], axis=0)


# speedup vs baseline: 4.7507x; 1.1183x over previous
import jax
import jax.numpy as jnp
from jax import lax
from jax.experimental import pallas as pl
from jax.experimental.pallas import tpu as pltpu
from jax.experimental.pallas import tpu_sc as plsc

N_NODE = 10000
E = 160000
BASIS_UNITS = 4
MSG_RED = 64
MSG = 192
MSGH = 96
MSG_FULL = 384
OUT_UNITS = 256

NC = 2
NS = 16
NW = NC * NS
EPC = E // NC
EPW = E // NW
CHUNK = 128
NFULL = EPW // CHUNK
TAIL = EPW - NFULL * CHUNK
ROWS_PER_S = N_NODE // NS
GCH = 80
NGCH = 128
N_PAD = NGCH * GCH

def _sc_mesh():
    return plsc.VectorSubcoreMesh(core_axis_name="c", subcore_axis_name="s",
                                  num_cores=NC, num_subcores=NS)


def _wfull_body(att_ref, basis_ref, out_ref):
    a = att_ref[...]
    b = basis_ref[...]
    for r in range(2):
        acc = a[r, 0] * b[0]
        for k in range(1, BASIS_UNITS):
            acc = acc + a[r, k] * b[k]
        out_ref[r] = acc


def _wfull(att, basis):
    nblk = 10
    blk = N_NODE // nblk
    return pl.pallas_call(
        _wfull_body,
        grid=(nblk,),
        in_specs=[
            pl.BlockSpec((2, BASIS_UNITS), lambda i: (0, 0)),
            pl.BlockSpec((BASIS_UNITS, blk, MSG_RED), lambda i: (0, i, 0)),
        ],
        out_specs=pl.BlockSpec((2, blk, MSG_RED), lambda i: (0, i, 0)),
        out_shape=jax.ShapeDtypeStruct((2, N_NODE, MSG_RED), jnp.float32),
    )(att, basis)


def _pre_body(s1m, s1t, d1m, d1t, s2m, s2t, d2m, d2t,
              dc0, dc1, dc2, ic0, ic1, ic2,
              w0, w1, ones8, zeros8,
              degd_p, degi_p, g_drug, g_dis,
              dacc, iacc, idxbig, idx_v8, ones_v, gbufa, gbufb, idx4,
              sema, semb):
    c = lax.axis_index("c")
    s = lax.axis_index("s")
    w = c * NS + s
    row0 = s * ROWS_PER_S

    pltpu.sync_copy(zeros8, dacc.at[pl.ds(row0, ROWS_PER_S)])
    pltpu.sync_copy(zeros8, iacc.at[pl.ds(row0, ROWS_PER_S)])
    pltpu.sync_copy(ones8, ones_v)
    plsc.subcore_barrier()

    for sm, st, acc in ((s1m, s1t, dacc), (s2m, s2t, dacc),
                        (d1m, d1t, iacc), (d2m, d2t, iacc)):
        pltpu.sync_copy(sm.at[w], idxbig)
        def deg_chunk(j, acc=acc):
            pltpu.sync_copy(ones_v, acc.at[idxbig.at[j]], add=True)
        lax.fori_loop(0, NFULL, lambda j, _, f=deg_chunk: (f(j), 0)[1], 0)
        pltpu.sync_copy(st.at[w], idx_v8)
        pltpu.sync_copy(ones_v.at[pl.ds(0, TAIL)], acc.at[idx_v8], add=True)

    plsc.subcore_barrier()
    pltpu.sync_copy(dacc.at[pl.ds(row0, ROWS_PER_S)], degd_p.at[c, s])
    pltpu.sync_copy(iacc.at[pl.ds(row0, ROWS_PER_S)], degi_p.at[c, s])

    wtabs = (w0, w1)
    bufs, sems = (gbufa, gbufb), (sema, semb)
    pend = []
    q = 0

    def drain():
        buf, sem, gout, r, k, t = pend.pop()
        pltpu.make_async_copy(wtabs[0].at[idx4.at[0]], buf, sem).wait()
        nb = pl.multiple_of((w + NW * t) * GCH, 8)
        pltpu.sync_copy(buf, gout.at[r, k, pl.ds(nb, GCH)])

    for side, cols, gout in ((0, (dc0, dc1, dc2), g_drug),
                             (1, (ic0, ic1, ic2), g_dis)):
        for k in range(3):
            if pend:
                drain()
            pltpu.sync_copy(cols[k].at[:, w], idx4)
            for r in range(2):
                for t in range(4):
                    b = q % 2
                    pltpu.async_copy(wtabs[r].at[idx4.at[t]], bufs[b],
                                     sems[b])
                    if pend:
                        drain()
                    pend.append((bufs[b], sems[b], gout, r, k, t))
                    q += 1
    drain()


def _pre(edges_split, dcols, icols, w0, w1):
    ones8 = jnp.ones((CHUNK, 8), jnp.float32)
    zeros8 = jnp.zeros((ROWS_PER_S, 8), jnp.float32)
    out_type = (
        jax.ShapeDtypeStruct((NC, NS, ROWS_PER_S, 8), jnp.float32),
        jax.ShapeDtypeStruct((NC, NS, ROWS_PER_S, 8), jnp.float32),
        jax.ShapeDtypeStruct((2, 3, N_PAD, MSG_RED), jnp.float32),
        jax.ShapeDtypeStruct((2, 3, N_PAD, MSG_RED), jnp.float32),
    )
    scratch = [
        pltpu.VMEM_SHARED((N_NODE, 8), jnp.float32),
        pltpu.VMEM_SHARED((N_NODE, 8), jnp.float32),
        pltpu.VMEM((NFULL, CHUNK), jnp.int32),
        pltpu.VMEM((TAIL,), jnp.int32),
        pltpu.VMEM((CHUNK, 8), jnp.float32),
        pltpu.VMEM((GCH, MSG_RED), jnp.float32),
        pltpu.VMEM((GCH, MSG_RED), jnp.float32),
        pltpu.VMEM((4, GCH), jnp.int32),
        pltpu.SemaphoreType.DMA,
        pltpu.SemaphoreType.DMA,
    ]
    f = pl.kernel(_pre_body, out_type=out_type, mesh=_sc_mesh(),
                  scratch_types=scratch,
                  compiler_params=pltpu.CompilerParams(use_tc_tiling_on_sc=False))
    return f(*edges_split, dcols[0], dcols[1], dcols[2],
             icols[0], icols[1], icols[2], w0, w1, ones8, zeros8)


def _scale_body(degd_ref, degi_ref, gd_ref, gi_ref,
                f1d_ref, f2d_ref, f1i_ref, f2i_ref, cd_ref, ci_ref):
    dd = degd_ref[0, :, 0] + degd_ref[1, :, 0]
    di = degi_ref[0, :, 0] + degi_ref[1, :, 0]
    cd = lax.rsqrt(jnp.maximum(dd, 1.0))[:, None]
    ci = lax.rsqrt(jnp.maximum(di, 1.0))[:, None]
    cd_ref[...] = cd
    ci_ref[...] = ci
    frefs = ((f1d_ref, f1i_ref), (f2d_ref, f2i_ref))
    for r in range(2):
        fd = jnp.concatenate(
            [gd_ref[r, 0], gd_ref[r, 1], gd_ref[r, 2]], axis=1) * cd
        fi = jnp.concatenate(
            [gi_ref[r, 0], gi_ref[r, 1], gi_ref[r, 2]], axis=1) * ci
        frefs[r][0][...] = fd.astype(jnp.bfloat16)
        frefs[r][1][...] = fi.astype(jnp.bfloat16)


def _scale(degd_p, degi_p, g_drug, g_dis):
    nblk = 10
    blk = N_NODE // nblk
    out_type = tuple(
        jax.ShapeDtypeStruct((N_NODE, MSG), jnp.bfloat16) for _ in range(4)
    ) + (jax.ShapeDtypeStruct((N_NODE, 1), jnp.float32),
         jax.ShapeDtypeStruct((N_NODE, 1), jnp.float32))
    deg_spec = pl.BlockSpec((NC, blk, 8), lambda i: (0, i, 0))
    g_spec = pl.BlockSpec((2, 3, blk, MSG_RED), lambda i: (0, 0, i, 0))
    f_spec = pl.BlockSpec((blk, MSG), lambda i: (i, 0))
    c_spec = pl.BlockSpec((blk, 1), lambda i: (i, 0))
    return pl.pallas_call(
        _scale_body,
        grid=(nblk,),
        in_specs=[deg_spec, deg_spec, g_spec, g_spec],
        out_specs=[f_spec] * 4 + [c_spec, c_spec],
        out_shape=out_type,
    )(degd_p, degi_p, g_drug, g_dis)


def _conv_body(f1d, f2d, f1i, f2i,
               s1m, s1t, d1m, d1t, s2m, s2t, d2m, d2t, zeros125,
               rst_p, acc, sidx_all, didx_all, sidx8, didx8,
               msga, msgb, msg8, zbuf, sema, semb):
    c = lax.axis_index("c")
    s = lax.axis_index("s")
    w = c * NS + s
    row0 = s * ROWS_PER_S

    pltpu.sync_copy(zeros125, zbuf)

    convs = ((f1d, s1m, s1t, d1m, d1t),
             (f2d, s2m, s2t, d2m, d2t),
             (f1i, d1m, d1t, s1m, s1t),
             (f2i, d2m, d2t, s2m, s2t))
    for ci, (ftab, sm, st, dm, dt) in enumerate(convs):
        pltpu.sync_copy(sm.at[w], sidx_all)
        pltpu.sync_copy(dm.at[w], didx_all)
        pltpu.sync_copy(st.at[w], sidx8)
        pltpu.sync_copy(dt.at[w], didx8)
        for j in range(5):
            pltpu.sync_copy(zbuf, acc.at[pl.ds(row0 + j * 125, 125)])
        plsc.subcore_barrier()

        def start_g(j, buf, sem, ftab=ftab):
            pltpu.async_copy(ftab.at[sidx_all.at[j]], buf, sem)

        def wait_g(buf, sem, ftab=ftab):
            pltpu.make_async_copy(ftab.at[sidx_all.at[0]], buf, sem).wait()

        def scat(j, buf):
            pltpu.sync_copy(buf, acc.at[didx_all.at[j]], add=True)

        start_g(0, msga, sema)

        def pair(j2, _):
            ja = 2 * j2
            wait_g(msga, sema)
            start_g(ja + 1, msgb, semb)
            scat(ja, msga)
            wait_g(msgb, semb)
            start_g(ja + 2, msga, sema)
            scat(ja + 1, msgb)
            return 0
        lax.fori_loop(0, (NFULL - 1) // 2, pair, 0)
        wait_g(msga, sema)
        scat(NFULL - 1, msga)
        pltpu.async_copy(ftab.at[sidx8], msg8, sema).wait()
        pltpu.sync_copy(msg8, acc.at[didx8], add=True)

        plsc.subcore_barrier()
        pltpu.sync_copy(acc.at[pl.ds(row0, ROWS_PER_S)],
                        rst_p.at[ci, c, s])
        plsc.subcore_barrier()


def _conv(ftabs, edges_split):
    zeros125 = jnp.zeros((125, MSG), jnp.bfloat16)
    out_type = jax.ShapeDtypeStruct((4, NC, NS, ROWS_PER_S, MSG),
                                    jnp.bfloat16)
    scratch = [
        pltpu.VMEM_SHARED((N_NODE, MSG), jnp.bfloat16),
        pltpu.VMEM((NFULL, CHUNK), jnp.int32),
        pltpu.VMEM((NFULL, CHUNK), jnp.int32),
        pltpu.VMEM((TAIL,), jnp.int32),
        pltpu.VMEM((TAIL,), jnp.int32),
        pltpu.VMEM((CHUNK, MSG), jnp.bfloat16),
        pltpu.VMEM((CHUNK, MSG), jnp.bfloat16),
        pltpu.VMEM((TAIL, MSG), jnp.bfloat16),
        pltpu.VMEM((125, MSG), jnp.bfloat16),
        pltpu.SemaphoreType.DMA,
        pltpu.SemaphoreType.DMA,
    ]
    f = pl.kernel(_conv_body, out_type=out_type, mesh=_sc_mesh(),
                  scratch_types=scratch,
                  compiler_params=pltpu.CompilerParams(use_tc_tiling_on_sc=False))
    return f(*ftabs, *edges_split, zeros125)


def _proj_body(pa_ref, pb_ref, c_ref, w1_ref, w2_ref, b_ref, out_ref):
    cc = c_ref[...]

    def m(ref):
        return (ref[0, 0].astype(jnp.float32)
                + ref[0, 1].astype(jnp.float32))
    h1 = m(pa_ref) * cc
    h2 = m(pb_ref) * cc
    out_ref[...] = (
        jnp.dot(h1, w1_ref[...], preferred_element_type=jnp.float32)
        + jnp.dot(h2, w2_ref[...], preferred_element_type=jnp.float32)
        + b_ref[...])


def _proj(rst_p, ca, cb, c_side, w1, w2, b2d):
    nblk = 10
    blk = N_NODE // nblk

    def p_spec(ci):
        return pl.BlockSpec((1, NC, blk, MSG),
                            lambda i, ci=ci: (ci, 0, i, 0))
    return pl.pallas_call(
        _proj_body,
        grid=(nblk,),
        in_specs=[
            p_spec(ca), p_spec(cb),
            pl.BlockSpec((blk, 1), lambda i: (i, 0)),
            pl.BlockSpec((MSG, OUT_UNITS), lambda i: (0, 0)),
            pl.BlockSpec((MSG, OUT_UNITS), lambda i: (0, 0)),
            pl.BlockSpec((1, OUT_UNITS), lambda i: (0, 0)),
        ],
        out_specs=pl.BlockSpec((blk, OUT_UNITS), lambda i: (i, 0)),
        out_shape=jax.ShapeDtypeStruct((N_NODE, OUT_UNITS), jnp.float32),
    )(rst_p, rst_p, c_side, w1, w2, b2d)


def kernel(drug_feat, dis_feat, edge_index_r1, edge_index_r2,
           att, basis, W_ufc, b_ufc):
    def split(e):
        m = e.astype(jnp.int32).reshape(NW, EPW)
        return (m[:, :NFULL * CHUNK].reshape(NW, NFULL, CHUNK),
                m[:, NFULL * CHUNK:])

    edges_split = []
    for arr in (edge_index_r1, edge_index_r2):
        for row in (0, 1):
            edges_split.extend(split(arr[row]))

    def colpack(feat, k):
        col = jnp.pad(feat[:, k].astype(jnp.int32), (0, N_PAD - N_NODE))
        return col.reshape(4, NW, GCH)
    dcols = [colpack(drug_feat, k) for k in range(3)]
    icols = [colpack(dis_feat, k) for k in range(3)]

    wf = _wfull(att, basis)
    degd_p, degi_p, g_drug, g_dis = _pre(
        edges_split, dcols, icols, wf[0], wf[1])
    degd_p = degd_p.reshape(NC, N_NODE, 8)
    degi_p = degi_p.reshape(NC, N_NODE, 8)
    *ftabs, c_drug, c_dis = _scale(degd_p, degi_p, g_drug, g_dis)
    rst_p = _conv(ftabs, edges_split)
    rst_p = rst_p.reshape(4, NC, N_NODE, MSG)

    w1 = W_ufc[:MSG]
    w2 = W_ufc[MSG:]
    b2d = b_ufc[None, :]
    out_drug = _proj(rst_p, 2, 3, c_drug, w1, w2, b2d)
    out_dis = _proj(rst_p, 0, 1, c_dis, w1, w2, b2d)
    return jnp.concatenate([out_drug, out_dis], axis=0)


# speedup vs baseline: 6.0960x; 1.2832x over previous
import jax
import jax.numpy as jnp
from jax import lax
from jax.experimental import pallas as pl
from jax.experimental.pallas import tpu as pltpu
from jax.experimental.pallas import tpu_sc as plsc

N_NODE = 10000
E = 160000
BASIS_UNITS = 4
MSG_RED = 64
MSG = 192
MSGH = 96
MSG_FULL = 384
OUT_UNITS = 256

NC = 2
NS = 16
NW = NC * NS
EPC = E // NC
EPW = E // NW
CHUNK = 128
NFULL = EPW // CHUNK
TAIL = EPW - NFULL * CHUNK
ROWS_PER_S = N_NODE // NS
GCH = 80
NGCH = 128
N_PAD = NGCH * GCH

def _sc_mesh():
    return plsc.VectorSubcoreMesh(core_axis_name="c", subcore_axis_name="s",
                                  num_cores=NC, num_subcores=NS)


def _wfull_body(att_ref, basis_ref, out_ref):
    a = att_ref[...]
    b = basis_ref[...]
    for r in range(2):
        acc = a[r, 0] * b[0]
        for k in range(1, BASIS_UNITS):
            acc = acc + a[r, k] * b[k]
        out_ref[r] = acc.astype(jnp.bfloat16)


def _wfull(att, basis):
    nblk = 10
    blk = N_NODE // nblk
    return pl.pallas_call(
        _wfull_body,
        grid=(nblk,),
        in_specs=[
            pl.BlockSpec((2, BASIS_UNITS), lambda i: (0, 0)),
            pl.BlockSpec((BASIS_UNITS, blk, MSG_RED), lambda i: (0, i, 0)),
        ],
        out_specs=pl.BlockSpec((2, blk, MSG_RED), lambda i: (0, i, 0)),
        out_shape=jax.ShapeDtypeStruct((2, N_NODE, MSG_RED), jnp.bfloat16),
    )(att, basis)


def _ctab_body(degd_ref, degi_ref, cd_ref, ci_ref):
    dd = degd_ref[0] + degd_ref[1]
    di = degi_ref[0] + degi_ref[1]
    cd_ref[...] = lax.rsqrt(jnp.maximum(dd, 1.0))
    ci_ref[...] = lax.rsqrt(jnp.maximum(di, 1.0))


def _ctab(degd_p, degi_p):
    nblk = 10
    blk = N_NODE // nblk
    deg_spec = pl.BlockSpec((NC, blk, 8), lambda i: (0, i, 0))
    c_spec = pl.BlockSpec((blk, 8), lambda i: (i, 0))
    return pl.pallas_call(
        _ctab_body,
        grid=(nblk,),
        in_specs=[deg_spec, deg_spec],
        out_specs=[c_spec, c_spec],
        out_shape=(jax.ShapeDtypeStruct((N_PAD, 8), jnp.float32),
                   jax.ShapeDtypeStruct((N_PAD, 8), jnp.float32)),
    )(degd_p, degi_p)


def _deg_body(s1m, s1t, d1m, d1t, s2m, s2t, d2m, d2t, ones8, zeros8,
              degd_p, degi_p,
              dacc, iacc, idxbig, idx_v8, ones_v, sem):
    c = lax.axis_index("c")
    s = lax.axis_index("s")
    w = c * NS + s
    row0 = s * ROWS_PER_S

    pltpu.sync_copy(zeros8, dacc.at[pl.ds(row0, ROWS_PER_S)])
    pltpu.sync_copy(zeros8, iacc.at[pl.ds(row0, ROWS_PER_S)])
    pltpu.sync_copy(ones8, ones_v)
    plsc.subcore_barrier()

    for sm, st, acc in ((s1m, s1t, dacc), (s2m, s2t, dacc),
                        (d1m, d1t, iacc), (d2m, d2t, iacc)):
        pltpu.sync_copy(sm.at[w], idxbig)
        def deg_chunk(j, acc=acc):
            pltpu.sync_copy(ones_v, acc.at[idxbig.at[j]], add=True)
        lax.fori_loop(0, NFULL, lambda j, _, f=deg_chunk: (f(j), 0)[1], 0)
        pltpu.sync_copy(st.at[w], idx_v8)
        pltpu.sync_copy(ones_v.at[pl.ds(0, TAIL)], acc.at[idx_v8], add=True)

    plsc.subcore_barrier()
    pltpu.sync_copy(dacc.at[pl.ds(row0, ROWS_PER_S)], degd_p.at[c, s])
    pltpu.sync_copy(iacc.at[pl.ds(row0, ROWS_PER_S)], degi_p.at[c, s])
    del sem


def _deg(edges_split):
    ones8 = jnp.ones((CHUNK, 8), jnp.float32)
    zeros8 = jnp.zeros((ROWS_PER_S, 8), jnp.float32)
    out_type = (
        jax.ShapeDtypeStruct((NC, NS, ROWS_PER_S, 8), jnp.float32),
        jax.ShapeDtypeStruct((NC, NS, ROWS_PER_S, 8), jnp.float32),
    )
    scratch = [
        pltpu.VMEM_SHARED((N_NODE, 8), jnp.float32),
        pltpu.VMEM_SHARED((N_NODE, 8), jnp.float32),
        pltpu.VMEM((NFULL, CHUNK), jnp.int32),
        pltpu.VMEM((TAIL,), jnp.int32),
        pltpu.VMEM((CHUNK, 8), jnp.float32),
        pltpu.SemaphoreType.DMA,
    ]
    f = pl.kernel(_deg_body, out_type=out_type, mesh=_sc_mesh(),
                  scratch_types=scratch,
                  compiler_params=pltpu.CompilerParams(use_tc_tiling_on_sc=False))
    return f(*edges_split, ones8, zeros8)


def _fbuild_body(dc0, dc1, dc2, ic0, ic1, ic2, w0, w1, cd8, ci8,
                 f1d, f2d, f1i, f2i,
                 gbufa, gbufb, fbuf, cbuf, idx4, sema, semb):
    c = lax.axis_index("c")
    s = lax.axis_index("s")
    w = c * NS + s

    wtabs = (w0, w1)
    bufs, sems = (gbufa, gbufb), (sema, semb)
    pend = []
    q = 0
    lanes16 = lax.iota(jnp.int32, 16)

    def drain():
        buf, sem, fouts, r, k, t = pend.pop()
        pltpu.make_async_copy(wtabs[0].at[idx4.at[0]], buf, sem).wait()
        def row(i, _, t=t):
            cv = plsc.load_gather(
                cbuf, [lanes16 * 0 + (t * GCH + i), lanes16 * 0])
            cv2 = plsc.pack(cv, cv, format=plsc.PackFormat.INTERLEAVED)
            for j in range(2):
                x = buf[i, pl.ds(j * 32, 32)]
                fbuf[i, pl.ds(j * 32, 32)] = x * cv2
            return 0
        lax.fori_loop(0, GCH, row, 0)
        nb = pl.multiple_of((w + NW * t) * GCH, 8)
        pltpu.sync_copy(
            fbuf, fouts[r].at[pl.ds(nb, GCH), pl.ds(k * MSG_RED, MSG_RED)])

    for side, cols, ctab, fouts in ((0, (dc0, dc1, dc2), cd8, (f1d, f2d)),
                                    (1, (ic0, ic1, ic2), ci8, (f1i, f2i))):
        for k in range(3):
            if pend:
                drain()
            pltpu.sync_copy(cols[k].at[:, w], idx4)
            for t in range(4):
                nb = pl.multiple_of((w + NW * t) * GCH, 8)
                pltpu.sync_copy(ctab.at[pl.ds(nb, GCH)],
                                cbuf.at[pl.ds(t * GCH, GCH)])
            for r in range(2):
                for t in range(4):
                    b = q % 2
                    pltpu.async_copy(wtabs[r].at[idx4.at[t]], bufs[b],
                                     sems[b])
                    if pend:
                        drain()
                    pend.append((bufs[b], sems[b], fouts, r, k, t))
                    q += 1
    drain()


def _fbuild(dcols, icols, w0, w1, cd8, ci8):
    out_type = tuple(
        jax.ShapeDtypeStruct((N_PAD, MSG), jnp.bfloat16) for _ in range(4))
    scratch = [
        pltpu.VMEM((GCH, MSG_RED), jnp.bfloat16),
        pltpu.VMEM((GCH, MSG_RED), jnp.bfloat16),
        pltpu.VMEM((GCH, MSG_RED), jnp.bfloat16),
        pltpu.VMEM((4 * GCH, 8), jnp.float32),
        pltpu.VMEM((4, GCH), jnp.int32),
        pltpu.SemaphoreType.DMA,
        pltpu.SemaphoreType.DMA,
    ]
    f = pl.kernel(_fbuild_body, out_type=out_type, mesh=_sc_mesh(),
                  scratch_types=scratch,
                  compiler_params=pltpu.CompilerParams(
                      use_tc_tiling_on_sc=False, needs_layout_passes=False))
    return f(dcols[0], dcols[1], dcols[2], icols[0], icols[1], icols[2],
             w0, w1, cd8, ci8)


def _conv_body(f1d, f2d, f1i, f2i,
               s1m, s1t, d1m, d1t, s2m, s2t, d2m, d2t, zeros125,
               rst_p, acc, sidx_all, didx_all, sidx8, didx8,
               msga, msgb, msg8, zbuf, sema, semb):
    c = lax.axis_index("c")
    s = lax.axis_index("s")
    w = c * NS + s
    row0 = s * ROWS_PER_S

    pltpu.sync_copy(zeros125, zbuf)

    convs = ((f1d, s1m, s1t, d1m, d1t),
             (f2d, s2m, s2t, d2m, d2t),
             (f1i, d1m, d1t, s1m, s1t),
             (f2i, d2m, d2t, s2m, s2t))
    for ci, (ftab, sm, st, dm, dt) in enumerate(convs):
        pltpu.sync_copy(sm.at[w], sidx_all)
        pltpu.sync_copy(dm.at[w], didx_all)
        pltpu.sync_copy(st.at[w], sidx8)
        pltpu.sync_copy(dt.at[w], didx8)
        for j in range(5):
            pltpu.sync_copy(zbuf, acc.at[pl.ds(row0 + j * 125, 125)])
        plsc.subcore_barrier()

        def start_g(j, buf, sem, ftab=ftab):
            pltpu.async_copy(ftab.at[sidx_all.at[j]], buf, sem)

        def wait_g(buf, sem, ftab=ftab):
            pltpu.make_async_copy(ftab.at[sidx_all.at[0]], buf, sem).wait()

        def scat(j, buf):
            pltpu.sync_copy(buf, acc.at[didx_all.at[j]], add=True)

        start_g(0, msga, sema)

        def pair(j2, _):
            ja = 2 * j2
            wait_g(msga, sema)
            start_g(ja + 1, msgb, semb)
            scat(ja, msga)
            wait_g(msgb, semb)
            start_g(ja + 2, msga, sema)
            scat(ja + 1, msgb)
            return 0
        lax.fori_loop(0, (NFULL - 1) // 2, pair, 0)
        wait_g(msga, sema)
        scat(NFULL - 1, msga)
        pltpu.async_copy(ftab.at[sidx8], msg8, sema).wait()
        pltpu.sync_copy(msg8, acc.at[didx8], add=True)

        plsc.subcore_barrier()
        pltpu.sync_copy(acc.at[pl.ds(row0, ROWS_PER_S)],
                        rst_p.at[ci, c, s])
        plsc.subcore_barrier()


def _conv(ftabs, edges_split):
    zeros125 = jnp.zeros((125, MSG), jnp.bfloat16)
    out_type = jax.ShapeDtypeStruct((4, NC, NS, ROWS_PER_S, MSG),
                                    jnp.bfloat16)
    scratch = [
        pltpu.VMEM_SHARED((N_NODE, MSG), jnp.bfloat16),
        pltpu.VMEM((NFULL, CHUNK), jnp.int32),
        pltpu.VMEM((NFULL, CHUNK), jnp.int32),
        pltpu.VMEM((TAIL,), jnp.int32),
        pltpu.VMEM((TAIL,), jnp.int32),
        pltpu.VMEM((CHUNK, MSG), jnp.bfloat16),
        pltpu.VMEM((CHUNK, MSG), jnp.bfloat16),
        pltpu.VMEM((TAIL, MSG), jnp.bfloat16),
        pltpu.VMEM((125, MSG), jnp.bfloat16),
        pltpu.SemaphoreType.DMA,
        pltpu.SemaphoreType.DMA,
    ]
    f = pl.kernel(_conv_body, out_type=out_type, mesh=_sc_mesh(),
                  scratch_types=scratch,
                  compiler_params=pltpu.CompilerParams(use_tc_tiling_on_sc=False))
    return f(*ftabs, *edges_split, zeros125)


def _proj_body(pa_ref, pb_ref, c_ref, w1_ref, w2_ref, b_ref, out_ref):
    cc = c_ref[:, 0:1]

    def m(ref):
        return (ref[0, 0].astype(jnp.float32)
                + ref[0, 1].astype(jnp.float32))
    h1 = m(pa_ref) * cc
    h2 = m(pb_ref) * cc
    out_ref[...] = (
        jnp.dot(h1, w1_ref[...], preferred_element_type=jnp.float32)
        + jnp.dot(h2, w2_ref[...], preferred_element_type=jnp.float32)
        + b_ref[...])


def _proj(rst_p, ca, cb, c_side, w1, w2, b2d):
    nblk = 10
    blk = N_NODE // nblk

    def p_spec(ci):
        return pl.BlockSpec((1, NC, blk, MSG),
                            lambda i, ci=ci: (ci, 0, i, 0))
    return pl.pallas_call(
        _proj_body,
        grid=(nblk,),
        in_specs=[
            p_spec(ca), p_spec(cb),
            pl.BlockSpec((blk, 8), lambda i: (i, 0)),
            pl.BlockSpec((MSG, OUT_UNITS), lambda i: (0, 0)),
            pl.BlockSpec((MSG, OUT_UNITS), lambda i: (0, 0)),
            pl.BlockSpec((1, OUT_UNITS), lambda i: (0, 0)),
        ],
        out_specs=pl.BlockSpec((blk, OUT_UNITS), lambda i: (i, 0)),
        out_shape=jax.ShapeDtypeStruct((N_NODE, OUT_UNITS), jnp.float32),
    )(rst_p, rst_p, c_side, w1, w2, b2d)


def kernel(drug_feat, dis_feat, edge_index_r1, edge_index_r2,
           att, basis, W_ufc, b_ufc):
    def split(e):
        m = e.astype(jnp.int32).reshape(NW, EPW)
        return (m[:, :NFULL * CHUNK].reshape(NW, NFULL, CHUNK),
                m[:, NFULL * CHUNK:])

    edges_split = []
    for arr in (edge_index_r1, edge_index_r2):
        for row in (0, 1):
            edges_split.extend(split(arr[row]))

    def colpack(feat, k):
        col = jnp.pad(feat[:, k].astype(jnp.int32), (0, N_PAD - N_NODE))
        return col.reshape(4, NW, GCH)
    dcols = [colpack(drug_feat, k) for k in range(3)]
    icols = [colpack(dis_feat, k) for k in range(3)]

    wf = _wfull(att, basis)
    degd_p, degi_p = _deg(edges_split)
    degd_p = degd_p.reshape(NC, N_NODE, 8)
    degi_p = degi_p.reshape(NC, N_NODE, 8)
    cd8, ci8 = _ctab(degd_p, degi_p)
    ftabs = _fbuild(dcols, icols, wf[0], wf[1], cd8, ci8)
    rst_p = _conv(ftabs, edges_split)
    rst_p = rst_p.reshape(4, NC, N_NODE, MSG)

    w1 = W_ufc[:MSG]
    w2 = W_ufc[MSG:]
    b2d = b_ufc[None, :]
    out_drug = _proj(rst_p, 2, 3, cd8, w1, w2, b2d)
    out_dis = _proj(rst_p, 0, 1, ci8, w1, w2, b2d)
    return jnp.concatenate([out_drug, out_dis], axis=0)
